# Initial kernel scaffold; baseline (speedup 1.0000x reference)
import jax
import jax.numpy as jnp
from jax.experimental import pallas as pl


def kernel(quadrant_ids, stance_consistency, emb_table, Wq, bq, Wf, bf, ln_g, ln_b):
    raise NotImplementedError("write your pallas kernel here")


# speedup vs baseline: 3.9968x; 3.9968x over previous
import jax
import jax.numpy as jnp
from jax.experimental import pallas as pl
from jax.experimental.pallas import tpu as pltpu

_NQ = 4
_E = 128
_O = 256
_TOK = 2048


def _fused_body(ids_ref, st_ref, emb_ref, wstack_ref, wf1_ref, wf2_ref,
                bf_ref, g_ref, b_ref, out_ref):
    q = jnp.clip(ids_ref[...] - 1, 0, _NQ - 1)
    lanes = jax.lax.broadcasted_iota(jnp.int32, (1, _NQ), 1)
    a = (q == lanes).astype(jnp.float32)
    s0 = st_ref[:, 0:1]
    s1 = st_ref[:, 1:2]
    feats = jnp.concatenate([a * s0, a * s1, a], axis=1)
    pre = jnp.dot(feats, wstack_ref[...],
                  preferred_element_type=jnp.float32)
    x = jnp.maximum(pre, 0.0)
    embp = jnp.dot(emb_ref[...], wf1_ref[...],
                   preferred_element_type=jnp.float32) + bf_ref[...]
    h = (jnp.dot(x, wf2_ref[...], preferred_element_type=jnp.float32)
         + jnp.dot(a, embp, preferred_element_type=jnp.float32))
    mu = jnp.mean(h, axis=-1, keepdims=True)
    d = h - mu
    var = jnp.mean(d * d, axis=-1, keepdims=True)
    hn = d * jax.lax.rsqrt(var + 1e-5)
    out_ref[...] = jnp.maximum(hn * g_ref[...] + b_ref[...], 0.0)


def kernel(quadrant_ids, stance_consistency, emb_table, Wq, bq, Wf, bf, ln_g, ln_b):
    B, K = quadrant_ids.shape
    n = B * K
    ids = quadrant_ids.astype(jnp.int32).reshape(n, 1)
    st = stance_consistency.reshape(n, 2)
    wstack = jnp.concatenate([Wq[:, 0, :], Wq[:, 1, :], bq], axis=0)
    wf1 = Wf[:_E, :]
    wf2 = Wf[_E:, :]
    bf2 = bf.reshape(1, _O)
    g2 = ln_g.reshape(1, _O)
    b2 = ln_b.reshape(1, _O)

    grid = (n // _TOK,)
    out = pl.pallas_call(
        _fused_body,
        grid=grid,
        in_specs=[
            pl.BlockSpec((_TOK, 1), lambda i: (i, 0)),
            pl.BlockSpec((_TOK, 2), lambda i: (i, 0)),
            pl.BlockSpec((_NQ, _E), lambda i: (0, 0)),
            pl.BlockSpec((3 * _NQ, _E), lambda i: (0, 0)),
            pl.BlockSpec((_E, _O), lambda i: (0, 0)),
            pl.BlockSpec((_E, _O), lambda i: (0, 0)),
            pl.BlockSpec((1, _O), lambda i: (0, 0)),
            pl.BlockSpec((1, _O), lambda i: (0, 0)),
            pl.BlockSpec((1, _O), lambda i: (0, 0)),
        ],
        out_specs=pl.BlockSpec((_TOK, _O), lambda i: (i, 0)),
        out_shape=jax.ShapeDtypeStruct((n, _O), jnp.float32),
        compiler_params=pltpu.CompilerParams(
            dimension_semantics=("arbitrary",),
        ),
    )(ids, st, emb_table, wstack, wf1, wf2, bf2, g2, b2)
    return out.reshape(B, K, _O)


# speedup vs baseline: 4.2205x; 1.0560x over previous
import jax
import jax.numpy as jnp
from jax.experimental import pallas as pl
from jax.experimental.pallas import tpu as pltpu

_NQ = 4
_E = 128
_O = 256
_TOK = 2048


def _fused_body(ids_ref, st_ref, emb_ref, wstack_ref, wf1_ref, wf2_ref,
                bf_ref, g_ref, b_ref, out_ref):
    q = jnp.clip(ids_ref[...] - 1, 0, _NQ - 1)
    lanes12 = jax.lax.broadcasted_iota(jnp.int32, (1, 3 * _NQ), 1)
    pos = jax.lax.rem(lanes12, _NQ)
    grp = jax.lax.div(lanes12, _NQ)
    s0 = st_ref[:, 0:1]
    s1 = st_ref[:, 1:2]
    mult = jnp.where(grp == 0, s0, jnp.where(grp == 1, s1, 1.0))
    feats = jnp.where(q == pos, mult, 0.0)
    a = feats[:, 2 * _NQ:3 * _NQ]
    pre = jnp.dot(feats, wstack_ref[...],
                  preferred_element_type=jnp.float32)
    x = jnp.maximum(pre, 0.0)
    embp = jnp.dot(emb_ref[...], wf1_ref[...],
                   preferred_element_type=jnp.float32) + bf_ref[...]
    embp = embp - jnp.mean(embp, axis=-1, keepdims=True)
    d = (jnp.dot(x, wf2_ref[...], preferred_element_type=jnp.float32)
         + jnp.dot(a, embp, preferred_element_type=jnp.float32))
    var = jnp.mean(d * d, axis=-1, keepdims=True)
    hn = d * jax.lax.rsqrt(var + 1e-5)
    out_ref[...] = jnp.maximum(hn * g_ref[...] + b_ref[...], 0.0)


def kernel(quadrant_ids, stance_consistency, emb_table, Wq, bq, Wf, bf, ln_g, ln_b):
    B, K = quadrant_ids.shape
    n = B * K
    ids = quadrant_ids.astype(jnp.int32).reshape(n, 1)
    st = stance_consistency.reshape(n, 2)
    wstack = jnp.concatenate([Wq[:, 0, :], Wq[:, 1, :], bq], axis=0)
    wf1 = Wf[:_E, :]
    wf2 = Wf[_E:, :]
    wf2 = wf2 - jnp.mean(wf2, axis=-1, keepdims=True)
    bf2 = bf.reshape(1, _O)
    g2 = ln_g.reshape(1, _O)
    b2 = ln_b.reshape(1, _O)

    grid = (n // _TOK,)
    out = pl.pallas_call(
        _fused_body,
        grid=grid,
        in_specs=[
            pl.BlockSpec((_TOK, 1), lambda i: (i, 0)),
            pl.BlockSpec((_TOK, 2), lambda i: (i, 0)),
            pl.BlockSpec((_NQ, _E), lambda i: (0, 0)),
            pl.BlockSpec((3 * _NQ, _E), lambda i: (0, 0)),
            pl.BlockSpec((_E, _O), lambda i: (0, 0)),
            pl.BlockSpec((_E, _O), lambda i: (0, 0)),
            pl.BlockSpec((1, _O), lambda i: (0, 0)),
            pl.BlockSpec((1, _O), lambda i: (0, 0)),
            pl.BlockSpec((1, _O), lambda i: (0, 0)),
        ],
        out_specs=pl.BlockSpec((_TOK, _O), lambda i: (i, 0)),
        out_shape=jax.ShapeDtypeStruct((n, _O), jnp.float32),
        compiler_params=pltpu.CompilerParams(
            dimension_semantics=("arbitrary",),
        ),
    )(ids, st, emb_table, wstack, wf1, wf2, bf2, g2, b2)
    return out.reshape(B, K, _O)


# speedup vs baseline: 4.3732x; 1.0362x over previous
import jax
import jax.numpy as jnp
from jax.experimental import pallas as pl
from jax.experimental.pallas import tpu as pltpu

_NQ = 4
_E = 128
_O = 256
_TOK = 4096


def _fused_body(ids_ref, st_ref, emb_ref, wstack_ref, wf1_ref, wf2_ref,
                bf_ref, g_ref, b_ref, out_ref):
    q = jnp.clip(ids_ref[...] - 1, 0, _NQ - 1)
    lanes12 = jax.lax.broadcasted_iota(jnp.int32, (1, 3 * _NQ), 1)
    pos = jax.lax.rem(lanes12, _NQ)
    grp = jax.lax.div(lanes12, _NQ)
    s0 = st_ref[:, 0:1]
    s1 = st_ref[:, 1:2]
    mult = jnp.where(grp == 0, s0, jnp.where(grp == 1, s1, 1.0))
    feats = jnp.where(q == pos, mult, 0.0)
    a = feats[:, 2 * _NQ:3 * _NQ]
    pre = jnp.dot(feats, wstack_ref[...],
                  preferred_element_type=jnp.float32)
    x = jnp.maximum(pre, 0.0)
    embp = jnp.dot(emb_ref[...], wf1_ref[...],
                   preferred_element_type=jnp.float32) + bf_ref[...]
    embp = embp - jnp.mean(embp, axis=-1, keepdims=True)
    d = (jnp.dot(x, wf2_ref[...], preferred_element_type=jnp.float32)
         + jnp.dot(a, embp, preferred_element_type=jnp.float32))
    var = jnp.mean(d * d, axis=-1, keepdims=True)
    hn = d * jax.lax.rsqrt(var + 1e-5)
    out_ref[...] = jnp.maximum(hn * g_ref[...] + b_ref[...], 0.0)


def kernel(quadrant_ids, stance_consistency, emb_table, Wq, bq, Wf, bf, ln_g, ln_b):
    B, K = quadrant_ids.shape
    n = B * K
    ids = quadrant_ids.astype(jnp.int32).reshape(n, 1)
    st = stance_consistency.reshape(n, 2)
    wstack = jnp.concatenate([Wq[:, 0, :], Wq[:, 1, :], bq], axis=0)
    wf1 = Wf[:_E, :]
    wf2 = Wf[_E:, :]
    wf2 = wf2 - jnp.mean(wf2, axis=-1, keepdims=True)
    bf2 = bf.reshape(1, _O)
    g2 = ln_g.reshape(1, _O)
    b2 = ln_b.reshape(1, _O)

    grid = (n // _TOK,)
    out = pl.pallas_call(
        _fused_body,
        grid=grid,
        in_specs=[
            pl.BlockSpec((_TOK, 1), lambda i: (i, 0)),
            pl.BlockSpec((_TOK, 2), lambda i: (i, 0)),
            pl.BlockSpec((_NQ, _E), lambda i: (0, 0)),
            pl.BlockSpec((3 * _NQ, _E), lambda i: (0, 0)),
            pl.BlockSpec((_E, _O), lambda i: (0, 0)),
            pl.BlockSpec((_E, _O), lambda i: (0, 0)),
            pl.BlockSpec((1, _O), lambda i: (0, 0)),
            pl.BlockSpec((1, _O), lambda i: (0, 0)),
            pl.BlockSpec((1, _O), lambda i: (0, 0)),
        ],
        out_specs=pl.BlockSpec((_TOK, _O), lambda i: (i, 0)),
        out_shape=jax.ShapeDtypeStruct((n, _O), jnp.float32),
        compiler_params=pltpu.CompilerParams(
            dimension_semantics=("arbitrary",),
        ),
    )(ids, st, emb_table, wstack, wf1, wf2, bf2, g2, b2)
    return out.reshape(B, K, _O)


# speedup vs baseline: 5.1857x; 1.1858x over previous
import jax
import jax.numpy as jnp
import numpy as np
from jax.experimental import pallas as pl
from jax.experimental.pallas import tpu as pltpu

_NQ = 4
_E = 128
_O = 256
_TOK = 4096

def _fused_body(u_ref, emb_ref, wstack_ref, wf1_ref, wf2_ref,
                bf_ref, g_ref, b_ref, out_ref):
    u = u_ref[...]
    rows = jax.lax.broadcasted_iota(jnp.int32, (4, 12), 0)
    cols = jax.lax.broadcasted_iota(jnp.int32, (4, 12), 1)
    pm = (rows == jax.lax.div(cols, _NQ)).astype(jnp.float32)
    pq = (rows == 3).astype(jnp.float32)
    pos = jax.lax.rem(
        jax.lax.broadcasted_iota(jnp.int32, (1, 12), 1), _NQ
    ).astype(jnp.float32)
    mult12 = jnp.dot(u, pm, preferred_element_type=jnp.float32)
    q12 = jnp.dot(u, pq, preferred_element_type=jnp.float32)
    onehot = jnp.abs(q12 - pos) < 0.5
    feats = jnp.where(onehot, mult12, 0.0)
    a = feats[:, 2 * _NQ:3 * _NQ]
    pre = jnp.dot(feats, wstack_ref[...],
                  preferred_element_type=jnp.float32)
    x = jnp.maximum(pre, 0.0)
    embp = jnp.dot(emb_ref[...], wf1_ref[...],
                   preferred_element_type=jnp.float32) + bf_ref[...]
    embp = embp - jnp.mean(embp, axis=-1, keepdims=True)
    d = (jnp.dot(x, wf2_ref[...], preferred_element_type=jnp.float32)
         + jnp.dot(a, embp, preferred_element_type=jnp.float32))
    var = jnp.dot(d * d, jnp.full((_O, 1), 1.0 / _O, jnp.float32),
                  preferred_element_type=jnp.float32)
    r = jax.lax.rsqrt(var + 1e-5)
    out_ref[...] = jnp.maximum(d * r * g_ref[...] + b_ref[...], 0.0)


def kernel(quadrant_ids, stance_consistency, emb_table, Wq, bq, Wf, bf, ln_g, ln_b):
    B, K = quadrant_ids.shape
    n = B * K
    qf = jnp.clip(quadrant_ids.astype(jnp.int32) - 1, 0, _NQ - 1)
    qf = qf.reshape(n, 1).astype(jnp.float32)
    st = stance_consistency.reshape(n, 2)
    u = jnp.concatenate([st, jnp.ones((n, 1), jnp.float32), qf], axis=1)
    wstack = jnp.concatenate([Wq[:, 0, :], Wq[:, 1, :], bq], axis=0)
    wf1 = Wf[:_E, :]
    wf2 = Wf[_E:, :]
    wf2 = wf2 - jnp.mean(wf2, axis=-1, keepdims=True)
    bf2 = bf.reshape(1, _O)
    g2 = ln_g.reshape(1, _O)
    b2 = ln_b.reshape(1, _O)

    grid = (n // _TOK,)
    out = pl.pallas_call(
        _fused_body,
        grid=grid,
        in_specs=[
            pl.BlockSpec((_TOK, 4), lambda i: (i, 0)),
            pl.BlockSpec((_NQ, _E), lambda i: (0, 0)),
            pl.BlockSpec((3 * _NQ, _E), lambda i: (0, 0)),
            pl.BlockSpec((_E, _O), lambda i: (0, 0)),
            pl.BlockSpec((_E, _O), lambda i: (0, 0)),
            pl.BlockSpec((1, _O), lambda i: (0, 0)),
            pl.BlockSpec((1, _O), lambda i: (0, 0)),
            pl.BlockSpec((1, _O), lambda i: (0, 0)),
        ],
        out_specs=pl.BlockSpec((_TOK, _O), lambda i: (i, 0)),
        out_shape=jax.ShapeDtypeStruct((n, _O), jnp.float32),
        compiler_params=pltpu.CompilerParams(
            dimension_semantics=("arbitrary",),
        ),
    )(u, emb_table, wstack, wf1, wf2, bf2, g2, b2)
    return out.reshape(B, K, _O)


# speedup vs baseline: 5.7143x; 1.1019x over previous
import jax
import jax.numpy as jnp
import numpy as np
from jax.experimental import pallas as pl
from jax.experimental.pallas import tpu as pltpu

_NQ = 4
_E = 128
_O = 256
_TOK = 4096

def _fused_body(u_ref, emb_ref, wstack_ref, wf1_ref, wf2_ref,
                bf_ref, g_ref, b_ref, out_ref):
    u = u_ref[...]
    rows = jax.lax.broadcasted_iota(jnp.int32, (4, 12), 0)
    cols = jax.lax.broadcasted_iota(jnp.int32, (4, 12), 1)
    pm = (rows == jax.lax.div(cols, _NQ)).astype(jnp.float32)
    pq = (rows == 3).astype(jnp.float32)
    pos = jax.lax.rem(
        jax.lax.broadcasted_iota(jnp.int32, (1, 12), 1), _NQ
    ).astype(jnp.float32)
    mult12 = jnp.dot(u, pm, preferred_element_type=jnp.float32)
    q12 = jnp.dot(u, pq, preferred_element_type=jnp.float32)
    onehot = jnp.abs(q12 - pos) < 0.5
    feats = jnp.where(onehot, mult12, 0.0)
    a = feats[:, 2 * _NQ:3 * _NQ]
    pre = jnp.dot(feats, wstack_ref[...],
                  preferred_element_type=jnp.float32)
    x = jnp.maximum(pre, 0.0)
    embp = jnp.dot(emb_ref[...], wf1_ref[...],
                   preferred_element_type=jnp.float32) + bf_ref[...]
    embp = embp - jnp.mean(embp, axis=-1, keepdims=True)
    xa = jnp.concatenate([x, a], axis=1)
    wcomb = jnp.concatenate([wf2_ref[...], embp], axis=0)
    d = jnp.dot(xa, wcomb, preferred_element_type=jnp.float32)
    var = jnp.dot(d * d, jnp.full((_O, 1), 1.0 / _O, jnp.float32),
                  preferred_element_type=jnp.float32)
    r = jax.lax.rsqrt(var + 1e-5)
    out_ref[...] = jnp.maximum(d * r * g_ref[...] + b_ref[...], 0.0)


def kernel(quadrant_ids, stance_consistency, emb_table, Wq, bq, Wf, bf, ln_g, ln_b):
    B, K = quadrant_ids.shape
    n = B * K
    qf = jnp.clip(quadrant_ids.astype(jnp.int32) - 1, 0, _NQ - 1)
    qf = qf.reshape(n, 1).astype(jnp.float32)
    st = stance_consistency.reshape(n, 2)
    u = jnp.concatenate([st, jnp.ones((n, 1), jnp.float32), qf], axis=1)
    wstack = jnp.concatenate([Wq[:, 0, :], Wq[:, 1, :], bq], axis=0)
    wf1 = Wf[:_E, :]
    wf2 = Wf[_E:, :]
    wf2 = wf2 - jnp.mean(wf2, axis=-1, keepdims=True)
    bf2 = bf.reshape(1, _O)
    g2 = ln_g.reshape(1, _O)
    b2 = ln_b.reshape(1, _O)

    grid = (n // _TOK,)
    out = pl.pallas_call(
        _fused_body,
        grid=grid,
        in_specs=[
            pl.BlockSpec((_TOK, 4), lambda i: (i, 0)),
            pl.BlockSpec((_NQ, _E), lambda i: (0, 0)),
            pl.BlockSpec((3 * _NQ, _E), lambda i: (0, 0)),
            pl.BlockSpec((_E, _O), lambda i: (0, 0)),
            pl.BlockSpec((_E, _O), lambda i: (0, 0)),
            pl.BlockSpec((1, _O), lambda i: (0, 0)),
            pl.BlockSpec((1, _O), lambda i: (0, 0)),
            pl.BlockSpec((1, _O), lambda i: (0, 0)),
        ],
        out_specs=pl.BlockSpec((_TOK, _O), lambda i: (i, 0)),
        out_shape=jax.ShapeDtypeStruct((n, _O), jnp.float32),
        compiler_params=pltpu.CompilerParams(
            dimension_semantics=("arbitrary",),
        ),
    )(u, emb_table, wstack, wf1, wf2, bf2, g2, b2)
    return out.reshape(B, K, _O)
